# baseline (speedup 1.0000x reference)
import functools

import jax
import jax.numpy as jnp
from jax import lax
from jax.experimental import pallas as pl
from jax.experimental.pallas import tpu as pltpu
from jax.experimental.pallas import tpu_sc as plsc

N = 10000
E = 320000
D = 128
G = 64
C = 10

NC, NS = 2, 16
NW = NC * NS
CHUNK = 125
K = 80
NPAD = 10240
RPT = NPAD // NS
ICH = 80
NCH = RPT // ICH
NBUF = 10
PF = 5

_mesh = plsc.VectorSubcoreMesh(core_axis_name="c", subcore_axis_name="s")


def _deg_body(dst_hbm, zeros_hbm, ones_hbm, out0_hbm, out1_hbm, dst_v, ones_v,
              stage_v, acc_sh, sem):
    del sem
    cid = lax.axis_index("c")
    sid = lax.axis_index("s")
    wid = cid * NS + sid
    r0 = sid * RPT
    pltpu.sync_copy(zeros_hbm.at[pl.ds(r0, RPT)], stage_v)
    pltpu.sync_copy(stage_v, acc_sh.at[pl.ds(r0, RPT)])
    pltpu.sync_copy(dst_hbm.at[wid], dst_v)
    pltpu.sync_copy(ones_hbm, ones_v)
    plsc.subcore_barrier()

    def step(j, carry):
        pltpu.sync_copy(ones_v, acc_sh.at[dst_v.at[j]], add=True)
        return carry

    lax.fori_loop(0, K, step, 0)
    plsc.subcore_barrier()
    pltpu.sync_copy(acc_sh.at[pl.ds(r0, RPT)], stage_v)

    @pl.when(cid == 0)
    def _():
        pltpu.sync_copy(stage_v, out0_hbm.at[pl.ds(r0, RPT)])

    @pl.when(cid == 1)
    def _():
        pltpu.sync_copy(stage_v, out1_hbm.at[pl.ds(r0, RPT)])


_deg_call = functools.partial(
    pl.kernel,
    out_type=(jax.ShapeDtypeStruct((NPAD,), jnp.float32),
              jax.ShapeDtypeStruct((NPAD,), jnp.float32)),
    mesh=_mesh,
    scratch_types=[
        pltpu.VMEM((K, CHUNK), jnp.int32),
        pltpu.VMEM((CHUNK,), jnp.float32),
        pltpu.VMEM((RPT,), jnp.float32),
        pltpu.VMEM_SHARED((NPAD,), jnp.float32),
        pltpu.SemaphoreType.DMA,
    ],
)(_deg_body)


def _edge_body(src_hbm, dst_hbm, g_hbm, out_hbm, src_v, dst_v, rows_v, g_sh,
               acc_sh, semg, sems):
    cid = lax.axis_index("c")
    sid = lax.axis_index("s")
    wid = cid * NS + sid
    r0 = sid * RPT
    gtab = g_hbm if g_sh is None else g_sh

    def wait_g(b):
        pltpu.make_async_copy(gtab.at[src_v.at[0]], rows_v.at[b],
                              semg.at[b]).wait()

    def wait_s(b):
        pltpu.make_async_copy(rows_v.at[b], acc_sh.at[dst_v.at[0]],
                              sems.at[b]).wait()

    for c in range(NCH):
        pltpu.async_copy(g_hbm.at[pl.ds(r0 + c * ICH, ICH)],
                         rows_v.at[c, pl.ds(0, ICH)], semg.at[c])
    pltpu.sync_copy(src_hbm.at[wid], src_v)
    pltpu.sync_copy(dst_hbm.at[wid], dst_v)
    for c in range(NCH):
        pltpu.make_async_copy(g_hbm.at[pl.ds(r0, ICH)],
                              rows_v.at[c, pl.ds(0, ICH)], semg.at[c]).wait()
        if g_sh is not None:
            pltpu.sync_copy(rows_v.at[c, pl.ds(0, ICH)],
                            g_sh.at[pl.ds(r0 + c * ICH, ICH)])
        pltpu.sync_copy(rows_v.at[c, pl.ds(0, ICH)],
                        acc_sh.at[pl.ds(r0 + c * ICH, ICH)])
    plsc.subcore_barrier()

    def fire_gather(j, b):
        pltpu.async_copy(gtab.at[src_v.at[j]], rows_v.at[b], semg.at[b])

    def fire_scatter(j, b):
        pltpu.async_copy(rows_v.at[b], acc_sh.at[dst_v.at[j]], sems.at[b],
                         add=True)

    for u in range(PF):
        fire_gather(u, u)
    for u in range(NBUF):
        wait_g(u)
        fire_scatter(u, u)
        if u >= PF:
            wait_s(u - PF)
        fire_gather(u + PF, (u + PF) % NBUF)

    def mid(i, carry):
        j0 = i * NBUF
        for u in range(NBUF):
            b = u
            wait_g(b)
            fire_scatter(j0 + u, b)
            bn = (u + PF) % NBUF
            wait_s(bn)
            fire_gather(j0 + u + PF, bn)
        return carry

    lax.fori_loop(1, K // NBUF - 1, mid, 0)
    j0 = K - NBUF
    for u in range(NBUF):
        wait_g(u)
        fire_scatter(j0 + u, u)
        if u < PF:
            wait_s((u + PF) % NBUF)
            fire_gather(j0 + u + PF, (u + PF) % NBUF)
    for b in range(NBUF):
        wait_s(b)
    plsc.subcore_barrier()
    for c in range(NCH):
        pltpu.sync_copy(acc_sh.at[pl.ds(r0 + c * ICH, ICH)],
                        rows_v.at[c, pl.ds(0, ICH)])
        pltpu.async_copy(rows_v.at[c, pl.ds(0, ICH)],
                         out_hbm.at[cid, pl.ds(r0 + c * ICH, ICH)], semg.at[c])
    for c in range(NCH):
        pltpu.make_async_copy(rows_v.at[c, pl.ds(0, ICH)],
                              out_hbm.at[cid, pl.ds(r0, ICH)],
                              semg.at[c]).wait()


def _make_edge_call(F, spmem_gather):
    scratch = [
        pltpu.VMEM((K, CHUNK), jnp.int32),
        pltpu.VMEM((K, CHUNK), jnp.int32),
        pltpu.VMEM((NBUF, CHUNK, F), jnp.float32),
    ]
    if spmem_gather:
        body = _edge_body
        scratch.append(pltpu.VMEM_SHARED((NPAD, F), jnp.float32))
    else:
        def body(src_hbm, dst_hbm, g_hbm, out_hbm, src_v, dst_v, rows_v,
                 acc_sh, semg, sems):
            _edge_body(src_hbm, dst_hbm, g_hbm, out_hbm, src_v, dst_v, rows_v,
                       None, acc_sh, semg, sems)
    scratch += [
        pltpu.VMEM_SHARED((NPAD, F), jnp.float32),
        pltpu.SemaphoreType.DMA((NBUF,)),
        pltpu.SemaphoreType.DMA((NBUF,)),
    ]
    return functools.partial(
        pl.kernel,
        out_type=jax.ShapeDtypeStruct((NC, NPAD, F), jnp.float32),
        mesh=_mesh,
        scratch_types=scratch,
        compiler_params=pltpu.CompilerParams(use_tc_tiling_on_sc=False),
    )(body)


_edge16 = _make_edge_call(16, True)
_edge32 = _make_edge_call(32, True)


def _tc_mm1_body(x_ref, w_ref, h_ref):
    h_ref[...] = jnp.dot(x_ref[...], w_ref[...],
                         preferred_element_type=jnp.float32)


def _tc_first_body(h_ref, d0_ref, d1_ref, g_ref, dinv_ref):
    deg = (d0_ref[...] + d1_ref[...] + 1.0).reshape(1, NPAD)
    dinv_col = jnp.transpose(lax.rsqrt(deg))
    dinv_ref[...] = dinv_col
    g_ref[...] = h_ref[...] * dinv_col


def _tc_mid_body(acc_ref, g_ref, dinv_ref, b_ref, w_ref, gout_ref):
    p = acc_ref[0] + acc_ref[1] - g_ref[...]
    dinv = dinv_ref[...]
    h = jnp.maximum(p * dinv + b_ref[...], 0.0)
    gout_ref[...] = (
        jnp.dot(h, w_ref[...], preferred_element_type=jnp.float32) * dinv
    )


def _tc_mid2_body(acc_ref, g_ref, dinv_ref, b_ref, w_ref, ga_ref, gb_ref):
    p = acc_ref[0] + acc_ref[1] - g_ref[...]
    dinv = dinv_ref[...]
    h = jnp.maximum(p * dinv + b_ref[...], 0.0)
    g3 = jnp.dot(h, w_ref[...], preferred_element_type=jnp.float32) * dinv
    ga_ref[...] = g3[:, :32]
    gb_ref[...] = g3[:, 32:]


def _tc_last_body(acca_ref, accb_ref, ga_ref, gb_ref, dinv_ref, b_ref,
                  batch_ref, wl_ref, bl_ref, out_ref):
    p = jnp.concatenate(
        [acca_ref[0] + acca_ref[1] - ga_ref[...],
         accb_ref[0] + accb_ref[1] - gb_ref[...]], axis=1
    )
    h = jnp.maximum(p * dinv_ref[...] + b_ref[...], 0.0)
    bid = batch_ref[...].reshape(1, NPAD)
    onehot_t = (lax.broadcasted_iota(jnp.int32, (G, 1), 0) == bid).astype(
        jnp.float32
    )
    sums = jnp.dot(onehot_t, h, preferred_element_type=jnp.float32)
    ones_col = jnp.ones((NPAD, 1), jnp.float32)
    cnts = jnp.dot(onehot_t, ones_col,
                   preferred_element_type=jnp.float32)
    pooled = sums / jnp.maximum(cnts, 1.0)
    out_ref[...] = (
        jnp.dot(pooled, wl_ref[...], preferred_element_type=jnp.float32)
        + bl_ref[...]
    )


def _tc_mm1(x_p, W1):
    return pl.pallas_call(
        _tc_mm1_body,
        out_shape=jax.ShapeDtypeStruct((NPAD, 16), jnp.float32),
    )(x_p, W1)


def _tc_first(h1, d0, d1):
    return pl.pallas_call(
        _tc_first_body,
        out_shape=(
            jax.ShapeDtypeStruct((NPAD, 16), jnp.float32),
            jax.ShapeDtypeStruct((NPAD, 1), jnp.float32),
        ),
    )(h1, d0, d1)


def _tc_mid(acc, g, dinv, b, W, fout):
    return pl.pallas_call(
        _tc_mid_body,
        out_shape=jax.ShapeDtypeStruct((NPAD, fout), jnp.float32),
    )(acc, g, dinv, b, W)


def _tc_mid2(acc, g, dinv, b, W):
    return pl.pallas_call(
        _tc_mid2_body,
        out_shape=(jax.ShapeDtypeStruct((NPAD, 32), jnp.float32),
                   jax.ShapeDtypeStruct((NPAD, 32), jnp.float32)),
    )(acc, g, dinv, b, W)


def _tc_last(acca, accb, ga, gb, dinv, b, batch_p, Wl, bl):
    return pl.pallas_call(
        _tc_last_body,
        out_shape=jax.ShapeDtypeStruct((G, C), jnp.float32),
    )(acca, accb, ga, gb, dinv, b, batch_p, Wl, bl)


def kernel(x, edge_index, batch, W1, b1, W2, b2, W3, b3, Wl, bl):
    src_p = edge_index[0].reshape(NW, K, CHUNK)
    dst_p = edge_index[1].reshape(NW, K, CHUNK)
    x_p = jnp.pad(x, ((0, NPAD - N), (0, 0)))
    batch_p = jnp.pad(batch, (0, NPAD - N), constant_values=G)
    zeros_n = jnp.zeros((NPAD,), jnp.float32)
    ones_c = jnp.ones((CHUNK,), jnp.float32)

    deg0, deg1 = _deg_call(dst_p, zeros_n, ones_c)
    h1 = _tc_mm1(x_p, W1)
    g1, dinv = _tc_first(h1, deg0, deg1)
    acc1 = _edge16(src_p, dst_p, g1)
    g2 = _tc_mid(acc1, g1, dinv, b1.reshape(1, 16), W2, 32)
    acc2 = _edge32(src_p, dst_p, g2)
    g3a, g3b = _tc_mid2(acc2, g2, dinv, b2.reshape(1, 32), W3)
    acc3a = _edge32(src_p, dst_p, g3a)
    acc3b = _edge32(src_p, dst_p, g3b)
    out = _tc_last(acc3a, acc3b, g3a, g3b, dinv, b3.reshape(1, 64), batch_p,
                   Wl, bl.reshape(1, C))
    return out.reshape(-1)


# speedup vs baseline: 1.0005x; 1.0005x over previous
import functools

import jax
import jax.numpy as jnp
from jax import lax
from jax.experimental import pallas as pl
from jax.experimental.pallas import tpu as pltpu
from jax.experimental.pallas import tpu_sc as plsc

N = 10000
E = 320000
D = 128
G = 64
C = 10

NC, NS = 2, 16
NW = NC * NS
CHUNK = 125
K = 80
NPAD = 10240
RPT = NPAD // NS
ICH = 80
NCH = RPT // ICH
NBUF = 10
PF = 5

_mesh = plsc.VectorSubcoreMesh(core_axis_name="c", subcore_axis_name="s")


def _deg_body(dst_hbm, zeros_hbm, ones_hbm, out0_hbm, out1_hbm, dst_v, ones_v,
              stage_v, acc_sh, sem):
    del sem
    cid = lax.axis_index("c")
    sid = lax.axis_index("s")
    wid = cid * NS + sid
    r0 = sid * RPT
    pltpu.sync_copy(zeros_hbm.at[pl.ds(r0, RPT)], stage_v)
    pltpu.sync_copy(stage_v, acc_sh.at[pl.ds(r0, RPT)])
    pltpu.sync_copy(dst_hbm.at[wid], dst_v)
    pltpu.sync_copy(ones_hbm, ones_v)
    plsc.subcore_barrier()

    def step(j, carry):
        pltpu.sync_copy(ones_v, acc_sh.at[dst_v.at[j]], add=True)
        return carry

    lax.fori_loop(0, K, step, 0)
    plsc.subcore_barrier()
    pltpu.sync_copy(acc_sh.at[pl.ds(r0, RPT)], stage_v)

    @pl.when(cid == 0)
    def _():
        pltpu.sync_copy(stage_v, out0_hbm.at[pl.ds(r0, RPT)])

    @pl.when(cid == 1)
    def _():
        pltpu.sync_copy(stage_v, out1_hbm.at[pl.ds(r0, RPT)])


_deg_call = functools.partial(
    pl.kernel,
    out_type=(jax.ShapeDtypeStruct((NPAD,), jnp.float32),
              jax.ShapeDtypeStruct((NPAD,), jnp.float32)),
    mesh=_mesh,
    scratch_types=[
        pltpu.VMEM((K, CHUNK), jnp.int32),
        pltpu.VMEM((CHUNK,), jnp.float32),
        pltpu.VMEM((RPT,), jnp.float32),
        pltpu.VMEM_SHARED((NPAD,), jnp.float32),
        pltpu.SemaphoreType.DMA,
    ],
)(_deg_body)


def _edge_body(src_hbm, dst_hbm, g_hbm, out_hbm, src_v, dst_v, rows_v, g_sh,
               acc_sh, semg, sems):
    cid = lax.axis_index("c")
    sid = lax.axis_index("s")
    wid = cid * NS + sid
    r0 = sid * RPT

    def wait_g(b):
        pltpu.make_async_copy(g_sh.at[src_v.at[0]], rows_v.at[b],
                              semg.at[b]).wait()

    def wait_s(b):
        pltpu.make_async_copy(rows_v.at[b], acc_sh.at[dst_v.at[0]],
                              sems.at[b]).wait()

    for c in range(NCH):
        pltpu.async_copy(g_hbm.at[pl.ds(r0 + c * ICH, ICH)],
                         rows_v.at[c, pl.ds(0, ICH)], semg.at[c])
    pltpu.sync_copy(src_hbm.at[wid], src_v)
    pltpu.sync_copy(dst_hbm.at[wid], dst_v)
    for c in range(NCH):
        pltpu.make_async_copy(g_hbm.at[pl.ds(r0, ICH)],
                              rows_v.at[c, pl.ds(0, ICH)], semg.at[c]).wait()
        pltpu.sync_copy(rows_v.at[c, pl.ds(0, ICH)],
                        g_sh.at[pl.ds(r0 + c * ICH, ICH)])
        pltpu.sync_copy(rows_v.at[c, pl.ds(0, ICH)],
                        acc_sh.at[pl.ds(r0 + c * ICH, ICH)])
    plsc.subcore_barrier()

    def fire_gather(j, b):
        pltpu.async_copy(g_sh.at[src_v.at[j]], rows_v.at[b], semg.at[b])

    def fire_scatter(j, b):
        pltpu.async_copy(rows_v.at[b], acc_sh.at[dst_v.at[j]], sems.at[b],
                         add=True)

    for u in range(PF):
        fire_gather(u, u)
    for u in range(NBUF):
        wait_g(u)
        fire_scatter(u, u)
        if u >= PF:
            wait_s(u - PF)
        fire_gather(u + PF, (u + PF) % NBUF)

    def mid(i, carry):
        j0 = i * NBUF
        for u in range(NBUF):
            b = u
            wait_g(b)
            fire_scatter(j0 + u, b)
            bn = (u + PF) % NBUF
            wait_s(bn)
            fire_gather(j0 + u + PF, bn)
        return carry

    lax.fori_loop(1, K // NBUF - 1, mid, 0)
    j0 = K - NBUF
    for u in range(NBUF):
        wait_g(u)
        fire_scatter(j0 + u, u)
        if u < PF:
            wait_s((u + PF) % NBUF)
            fire_gather(j0 + u + PF, (u + PF) % NBUF)
    for b in range(NBUF):
        wait_s(b)
    plsc.subcore_barrier()
    for c in range(NCH):
        pltpu.sync_copy(acc_sh.at[pl.ds(r0 + c * ICH, ICH)],
                        rows_v.at[c, pl.ds(0, ICH)])
        pltpu.async_copy(rows_v.at[c, pl.ds(0, ICH)],
                         out_hbm.at[cid, pl.ds(r0 + c * ICH, ICH)], semg.at[c])
    for c in range(NCH):
        pltpu.make_async_copy(rows_v.at[c, pl.ds(0, ICH)],
                              out_hbm.at[cid, pl.ds(r0, ICH)],
                              semg.at[c]).wait()


def _make_edge_call(F):
    return functools.partial(
        pl.kernel,
        out_type=jax.ShapeDtypeStruct((NC, NPAD, F), jnp.float32),
        mesh=_mesh,
        scratch_types=[
            pltpu.VMEM((K, CHUNK), jnp.int32),
            pltpu.VMEM((K, CHUNK), jnp.int32),
            pltpu.VMEM((NBUF, CHUNK, F), jnp.float32),
            pltpu.VMEM_SHARED((NPAD, F), jnp.float32),
            pltpu.VMEM_SHARED((NPAD, F), jnp.float32),
            pltpu.SemaphoreType.DMA((NBUF,)),
            pltpu.SemaphoreType.DMA((NBUF,)),
        ],
        compiler_params=pltpu.CompilerParams(use_tc_tiling_on_sc=False),
    )(_edge_body)


_edge16 = _make_edge_call(16)
_edge32 = _make_edge_call(32)


def _tc_mm1_body(x_ref, w_ref, h_ref):
    h_ref[...] = jnp.dot(x_ref[...], w_ref[...],
                         preferred_element_type=jnp.float32)


def _tc_first_body(h_ref, d0_ref, d1_ref, g_ref, dinv_ref):
    deg = (d0_ref[...] + d1_ref[...] + 1.0).reshape(1, NPAD)
    dinv_col = jnp.transpose(lax.rsqrt(deg))
    dinv_ref[...] = dinv_col
    g_ref[...] = h_ref[...] * dinv_col


def _tc_mid_body(acc_ref, g_ref, dinv_ref, b_ref, w_ref, gout_ref):
    p = acc_ref[0] + acc_ref[1] - g_ref[...]
    dinv = dinv_ref[...]
    h = jnp.maximum(p * dinv + b_ref[...], 0.0)
    gout_ref[...] = (
        jnp.dot(h, w_ref[...], preferred_element_type=jnp.float32) * dinv
    )


def _tc_mid2_body(acc_ref, g_ref, dinv_ref, b_ref, w_ref, ga_ref, gb_ref):
    p = acc_ref[0] + acc_ref[1] - g_ref[...]
    dinv = dinv_ref[...]
    h = jnp.maximum(p * dinv + b_ref[...], 0.0)
    g3 = jnp.dot(h, w_ref[...], preferred_element_type=jnp.float32) * dinv
    ga_ref[...] = g3[:, :32]
    gb_ref[...] = g3[:, 32:]


def _tc_last_body(acca_ref, accb_ref, ga_ref, gb_ref, dinv_ref, b_ref,
                  batch_ref, wl_ref, bl_ref, out_ref):
    p = jnp.concatenate(
        [acca_ref[0] + acca_ref[1] - ga_ref[...],
         accb_ref[0] + accb_ref[1] - gb_ref[...]], axis=1
    )
    h = jnp.maximum(p * dinv_ref[...] + b_ref[...], 0.0)
    bid = batch_ref[...].reshape(1, NPAD)
    onehot_t = (lax.broadcasted_iota(jnp.int32, (G, 1), 0) == bid).astype(
        jnp.float32
    )
    sums = jnp.dot(onehot_t, h, preferred_element_type=jnp.float32)
    ones_col = jnp.ones((NPAD, 1), jnp.float32)
    cnts = jnp.dot(onehot_t, ones_col,
                   preferred_element_type=jnp.float32)
    pooled = sums / jnp.maximum(cnts, 1.0)
    out_ref[...] = (
        jnp.dot(pooled, wl_ref[...], preferred_element_type=jnp.float32)
        + bl_ref[...]
    )


def _tc_mm1(x_p, W1):
    return pl.pallas_call(
        _tc_mm1_body,
        out_shape=jax.ShapeDtypeStruct((NPAD, 16), jnp.float32),
    )(x_p, W1)


def _tc_first(h1, d0, d1):
    return pl.pallas_call(
        _tc_first_body,
        out_shape=(
            jax.ShapeDtypeStruct((NPAD, 16), jnp.float32),
            jax.ShapeDtypeStruct((NPAD, 1), jnp.float32),
        ),
    )(h1, d0, d1)


def _tc_mid(acc, g, dinv, b, W, fout):
    return pl.pallas_call(
        _tc_mid_body,
        out_shape=jax.ShapeDtypeStruct((NPAD, fout), jnp.float32),
    )(acc, g, dinv, b, W)


def _tc_mid2(acc, g, dinv, b, W):
    return pl.pallas_call(
        _tc_mid2_body,
        out_shape=(jax.ShapeDtypeStruct((NPAD, 32), jnp.float32),
                   jax.ShapeDtypeStruct((NPAD, 32), jnp.float32)),
    )(acc, g, dinv, b, W)


def _tc_last(acca, accb, ga, gb, dinv, b, batch_p, Wl, bl):
    return pl.pallas_call(
        _tc_last_body,
        out_shape=jax.ShapeDtypeStruct((G, C), jnp.float32),
    )(acca, accb, ga, gb, dinv, b, batch_p, Wl, bl)


def kernel(x, edge_index, batch, W1, b1, W2, b2, W3, b3, Wl, bl):
    src_p = edge_index[0].reshape(NW, K, CHUNK)
    dst_p = edge_index[1].reshape(NW, K, CHUNK)
    x_p = jnp.pad(x, ((0, NPAD - N), (0, 0)))
    batch_p = jnp.pad(batch, (0, NPAD - N), constant_values=G)
    zeros_n = jnp.zeros((NPAD,), jnp.float32)
    ones_c = jnp.ones((CHUNK,), jnp.float32)

    deg0, deg1 = _deg_call(dst_p, zeros_n, ones_c)
    h1 = _tc_mm1(x_p, W1)
    g1, dinv = _tc_first(h1, deg0, deg1)
    acc1 = _edge16(src_p, dst_p, g1)
    g2 = _tc_mid(acc1, g1, dinv, b1.reshape(1, 16), W2, 32)
    acc2 = _edge32(src_p, dst_p, g2)
    g3a, g3b = _tc_mid2(acc2, g2, dinv, b2.reshape(1, 32), W3)
    acc3a = _edge32(src_p, dst_p, g3a)
    acc3b = _edge32(src_p, dst_p, g3b)
    out = _tc_last(acc3a, acc3b, g3a, g3b, dinv, b3.reshape(1, 64), batch_p,
                   Wl, bl.reshape(1, C))
    return out.reshape(-1)
